# baseline (speedup 1.0000x reference)
import functools

import jax
import jax.numpy as jnp
from jax import lax
from jax.experimental import pallas as pl
from jax.experimental.pallas import tpu as pltpu
from jax.experimental.pallas import tpu_sc as plsc

N = 10000
D = 128
DP = 144
E = 320000
NC = 2
NS = 16
NW = NC * NS
B = 128
SB = 8
EPW0 = 19456
EPW1 = 1024
EPAD = NS * (EPW0 + EPW1)
RPT = N // NS


def _sc_body(g_hbm, src_hbm, dst_hbm, out_hbm,
             srcb, dstb, rows0, rows1, acc, semg0, semg1):
    cid = lax.axis_index("c")
    sid = lax.axis_index("s")
    ebase = jnp.where(cid == 0, sid * EPW0, NS * EPW0 + sid * EPW1)
    nch = jnp.where(cid == 0, EPW0 // B, EPW1 // B)

    with jax.named_scope("zero_phase"):
        def zstore(t, _):
            r = t // (DP // 16)
            k = t % (DP // 16)
            rows0[r, pl.ds(k * 16, 16)] = jnp.zeros((16,), jnp.float32)
            return 0
        lax.fori_loop(0, B * (DP // 16), zstore, 0)
        base = sid * RPT

        def zcopy(k, _):
            pltpu.sync_copy(rows0, acc.at[pl.ds(base + k * B, B), :])
            return 0
        lax.fori_loop(0, RPT // B, zcopy, 0)
        pltpu.sync_copy(rows0.at[pl.ds(0, RPT % B)],
                        acc.at[pl.ds(base + (RPT // B) * B, RPT % B), :])
        plsc.subcore_barrier()

    def superchunk(s, _):
        sbase = ebase + s * (SB * B)
        pltpu.sync_copy(src_hbm.at[pl.ds(sbase, SB * B)], srcb)
        pltpu.sync_copy(dst_hbm.at[pl.ds(sbase, SB * B)], dstb)
        pltpu.async_copy(g_hbm.at[srcb.at[pl.ds(0, B)]], rows0, semg0)

        def pair(jj, _):
            for b, (rcur, rnxt, scur, snxt) in enumerate(
                    ((rows0, rows1, semg0, semg1),
                     (rows1, rows0, semg1, semg0))):
                j = jj * 2 + b
                if b == 0:
                    pltpu.async_copy(
                        g_hbm.at[srcb.at[pl.ds((j + 1) * B, B)]], rnxt, snxt)
                else:

                    @pl.when(j + 1 < SB)
                    def _():
                        pltpu.async_copy(
                            g_hbm.at[srcb.at[pl.ds((j + 1) * B, B)]],
                            rnxt, snxt)
                pltpu.make_async_copy(
                    g_hbm.at[srcb.at[pl.ds(j * B, B)]], rcur, scur).wait()
                pltpu.sync_copy(rcur, acc.at[dstb.at[pl.ds(j * B, B)]],
                                add=True)
            return 0
        lax.fori_loop(0, SB // 2, pair, 0)
        return 0
    with jax.named_scope("accumulate"):
        lax.fori_loop(0, nch // SB, superchunk, 0)
        plsc.subcore_barrier()

    with jax.named_scope("flush"):
        pltpu.sync_copy(acc.at[pl.ds(sid * RPT, RPT), :],
                        out_hbm.at[cid, pl.ds(sid * RPT, RPT), :])


_sc_aggregate = functools.partial(
    pl.kernel,
    out_type=jax.ShapeDtypeStruct((NC, N, DP), jnp.float32),
    mesh=plsc.VectorSubcoreMesh(core_axis_name="c", subcore_axis_name="s"),
    compiler_params=pltpu.CompilerParams(use_tc_tiling_on_sc=False),
    scratch_types=[
        pltpu.VMEM((SB * B,), jnp.int32),
        pltpu.VMEM((SB * B,), jnp.int32),
        pltpu.VMEM((B, DP), jnp.float32),
        pltpu.VMEM((B, DP), jnp.float32),
        pltpu.VMEM_SHARED((N + 8, DP), jnp.float32),
        pltpu.SemaphoreType.DMA,
        pltpu.SemaphoreType.DMA,
    ],
)(_sc_body)


R = 1000


def _tc_project_body(x_ref, wl_ref, g_ref):
    g = jnp.dot(x_ref[...], wl_ref[...], preferred_element_type=jnp.float32)
    g_ref[:, :D] = g
    e = (lax.broadcasted_iota(jnp.int32, (R, DP - D), 1) == 0)
    g_ref[:, D:] = e.astype(jnp.float32)


def _tc_combine_body(p_ref, h_ref, wr_ref, bl_ref, wl_next_ref,
                     out_ref, g_ref=None, *, relu, project_next):
    s = p_ref[0] + p_ref[1]
    num = s[:, :D]
    cnt = s[:, D:D + 1]
    inv = 1.0 / jnp.maximum(cnt, 1.0)
    t = num * inv + bl_ref[...] + jnp.dot(
        h_ref[...], wr_ref[...], preferred_element_type=jnp.float32)
    nrm = jnp.sqrt(jnp.sum(t * t, axis=1, keepdims=True))
    h = t / jnp.maximum(nrm, 1e-12)
    if relu:
        h = jnp.maximum(h, 0.0)
    out_ref[...] = h
    if project_next:
        g = jnp.dot(h, wl_next_ref[...], preferred_element_type=jnp.float32)
        g_ref[:, :D] = g
        e = (lax.broadcasted_iota(jnp.int32, (R, DP - D), 1) == 0)
        g_ref[:, D:] = e.astype(jnp.float32)


def _tc_project(x, wl):
    return pl.pallas_call(
        _tc_project_body,
        grid=(N // R,),
        in_specs=[
            pl.BlockSpec((R, D), lambda i: (i, 0)),
            pl.BlockSpec((D, D), lambda i: (0, 0)),
        ],
        out_specs=pl.BlockSpec((R, DP), lambda i: (i, 0)),
        out_shape=jax.ShapeDtypeStruct((N, DP), jnp.float32),
    )(x, wl)


def _tc_combine(p, h, wr, bl, wl_next, relu, project_next):
    body = functools.partial(_tc_combine_body, relu=relu,
                             project_next=project_next)
    out_shapes = [jax.ShapeDtypeStruct((N, D), jnp.float32)]
    out_specs = [pl.BlockSpec((R, D), lambda i: (i, 0))]
    if project_next:
        out_shapes.append(jax.ShapeDtypeStruct((N, DP), jnp.float32))
        out_specs.append(pl.BlockSpec((R, DP), lambda i: (i, 0)))
    return pl.pallas_call(
        body,
        grid=(N // R,),
        in_specs=[
            pl.BlockSpec((NC, R, DP), lambda i: (0, i, 0)),
            pl.BlockSpec((R, D), lambda i: (i, 0)),
            pl.BlockSpec((D, D), lambda i: (0, 0)),
            pl.BlockSpec((1, D), lambda i: (0, 0)),
            pl.BlockSpec((D, D), lambda i: (0, 0)),
        ],
        out_specs=out_specs,
        out_shape=out_shapes,
    )(p, h, wr, bl, wl_next)


def kernel(x, edge_index, Wl0, bl0, Wr0, Wl1, bl1, Wr1):
    src = edge_index[0]
    dst = edge_index[1]
    pad = EPAD - E
    src_p = jnp.concatenate([src, jnp.zeros((pad,), jnp.int32)])
    dst_p = jnp.concatenate([dst, jnp.full((pad,), N, jnp.int32)])
    bl0r = bl0.reshape(1, D)
    bl1r = bl1.reshape(1, D)

    g0 = _tc_project(x, Wl0)
    p0 = _sc_aggregate(g0, src_p, dst_p)
    h1, g1 = _tc_combine(p0, x, Wr0, bl0r, Wl1, relu=True, project_next=True)
    p1 = _sc_aggregate(g1, src_p, dst_p)
    (out,) = _tc_combine(p1, h1, Wr1, bl1r, Wl1, relu=False,
                         project_next=False)
    return out


# speedup vs baseline: 1.0017x; 1.0017x over previous
import functools

import jax
import jax.numpy as jnp
from jax import lax
from jax.experimental import pallas as pl
from jax.experimental.pallas import tpu as pltpu
from jax.experimental.pallas import tpu_sc as plsc

N = 10000
D = 128
DP = 144
E = 320000
NC = 2
NS = 16
NW = NC * NS
B = 128
SB = 10
EPW0 = 19200
EPW1 = 1280
EPAD = NS * (EPW0 + EPW1)
RPT = N // NS


def _sc_body(g_hbm, src_hbm, dst_hbm, out_hbm,
             srcb, dstb, rows0, rows1, acc, semg0, semg1):
    cid = lax.axis_index("c")
    sid = lax.axis_index("s")
    ebase = jnp.where(cid == 0, sid * EPW0, NS * EPW0 + sid * EPW1)
    nch = jnp.where(cid == 0, EPW0 // B, EPW1 // B)

    with jax.named_scope("zero_phase"):
        def zstore(t, _):
            r = t // (DP // 16)
            k = t % (DP // 16)
            rows0[r, pl.ds(k * 16, 16)] = jnp.zeros((16,), jnp.float32)
            return 0
        lax.fori_loop(0, B * (DP // 16), zstore, 0)
        base = sid * RPT

        def zcopy(k, _):
            pltpu.sync_copy(rows0, acc.at[pl.ds(base + k * B, B), :])
            return 0
        lax.fori_loop(0, RPT // B, zcopy, 0)
        pltpu.sync_copy(rows0.at[pl.ds(0, RPT % B)],
                        acc.at[pl.ds(base + (RPT // B) * B, RPT % B), :])
        plsc.subcore_barrier()

    def superchunk(s, _):
        sbase = ebase + s * (SB * B)
        pltpu.sync_copy(src_hbm.at[pl.ds(sbase, SB * B)], srcb)
        pltpu.sync_copy(dst_hbm.at[pl.ds(sbase, SB * B)], dstb)
        pltpu.async_copy(g_hbm.at[srcb.at[pl.ds(0, B)]], rows0, semg0)

        def pair(jj, _):
            for b, (rcur, rnxt, scur, snxt) in enumerate(
                    ((rows0, rows1, semg0, semg1),
                     (rows1, rows0, semg1, semg0))):
                j = jj * 2 + b
                if b == 0:
                    pltpu.async_copy(
                        g_hbm.at[srcb.at[pl.ds((j + 1) * B, B)]], rnxt, snxt)
                else:

                    @pl.when(j + 1 < SB)
                    def _():
                        pltpu.async_copy(
                            g_hbm.at[srcb.at[pl.ds((j + 1) * B, B)]],
                            rnxt, snxt)
                pltpu.make_async_copy(
                    g_hbm.at[srcb.at[pl.ds(j * B, B)]], rcur, scur).wait()
                pltpu.sync_copy(rcur, acc.at[dstb.at[pl.ds(j * B, B)]],
                                add=True)
            return 0
        lax.fori_loop(0, SB // 2, pair, 0)
        return 0
    with jax.named_scope("accumulate"):
        lax.fori_loop(0, nch // SB, superchunk, 0)
        plsc.subcore_barrier()

    with jax.named_scope("flush"):
        pltpu.sync_copy(acc.at[pl.ds(sid * RPT, RPT), :],
                        out_hbm.at[cid, pl.ds(sid * RPT, RPT), :])


_sc_aggregate = functools.partial(
    pl.kernel,
    out_type=jax.ShapeDtypeStruct((NC, N, DP), jnp.float32),
    mesh=plsc.VectorSubcoreMesh(core_axis_name="c", subcore_axis_name="s"),
    compiler_params=pltpu.CompilerParams(use_tc_tiling_on_sc=False),
    scratch_types=[
        pltpu.VMEM((SB * B,), jnp.int32),
        pltpu.VMEM((SB * B,), jnp.int32),
        pltpu.VMEM((B, DP), jnp.float32),
        pltpu.VMEM((B, DP), jnp.float32),
        pltpu.VMEM_SHARED((N + 8, DP), jnp.float32),
        pltpu.SemaphoreType.DMA,
        pltpu.SemaphoreType.DMA,
    ],
)(_sc_body)


R = 1000


def _tc_project_body(x_ref, wl_ref, g_ref):
    g = jnp.dot(x_ref[...], wl_ref[...], preferred_element_type=jnp.float32)
    g_ref[:, :D] = g
    e = (lax.broadcasted_iota(jnp.int32, (R, DP - D), 1) == 0)
    g_ref[:, D:] = e.astype(jnp.float32)


def _tc_combine_body(p_ref, h_ref, wr_ref, bl_ref, wl_next_ref,
                     out_ref, g_ref=None, *, relu, project_next):
    s = p_ref[0] + p_ref[1]
    num = s[:, :D]
    cnt = s[:, D:D + 1]
    inv = 1.0 / jnp.maximum(cnt, 1.0)
    t = num * inv + bl_ref[...] + jnp.dot(
        h_ref[...], wr_ref[...], preferred_element_type=jnp.float32)
    nrm = jnp.sqrt(jnp.sum(t * t, axis=1, keepdims=True))
    h = t / jnp.maximum(nrm, 1e-12)
    if relu:
        h = jnp.maximum(h, 0.0)
    out_ref[...] = h
    if project_next:
        g = jnp.dot(h, wl_next_ref[...], preferred_element_type=jnp.float32)
        g_ref[:, :D] = g
        e = (lax.broadcasted_iota(jnp.int32, (R, DP - D), 1) == 0)
        g_ref[:, D:] = e.astype(jnp.float32)


def _tc_project(x, wl):
    return pl.pallas_call(
        _tc_project_body,
        grid=(N // R,),
        in_specs=[
            pl.BlockSpec((R, D), lambda i: (i, 0)),
            pl.BlockSpec((D, D), lambda i: (0, 0)),
        ],
        out_specs=pl.BlockSpec((R, DP), lambda i: (i, 0)),
        out_shape=jax.ShapeDtypeStruct((N, DP), jnp.float32),
    )(x, wl)


def _tc_combine(p, h, wr, bl, wl_next, relu, project_next):
    body = functools.partial(_tc_combine_body, relu=relu,
                             project_next=project_next)
    out_shapes = [jax.ShapeDtypeStruct((N, D), jnp.float32)]
    out_specs = [pl.BlockSpec((R, D), lambda i: (i, 0))]
    if project_next:
        out_shapes.append(jax.ShapeDtypeStruct((N, DP), jnp.float32))
        out_specs.append(pl.BlockSpec((R, DP), lambda i: (i, 0)))
    return pl.pallas_call(
        body,
        grid=(N // R,),
        in_specs=[
            pl.BlockSpec((NC, R, DP), lambda i: (0, i, 0)),
            pl.BlockSpec((R, D), lambda i: (i, 0)),
            pl.BlockSpec((D, D), lambda i: (0, 0)),
            pl.BlockSpec((1, D), lambda i: (0, 0)),
            pl.BlockSpec((D, D), lambda i: (0, 0)),
        ],
        out_specs=out_specs,
        out_shape=out_shapes,
    )(p, h, wr, bl, wl_next)


def kernel(x, edge_index, Wl0, bl0, Wr0, Wl1, bl1, Wr1):
    src = edge_index[0]
    dst = edge_index[1]
    pad = EPAD - E
    src_p = jnp.concatenate([src, jnp.zeros((pad,), jnp.int32)])
    dst_p = jnp.concatenate([dst, jnp.full((pad,), N, jnp.int32)])
    bl0r = bl0.reshape(1, D)
    bl1r = bl1.reshape(1, D)

    g0 = _tc_project(x, Wl0)
    p0 = _sc_aggregate(g0, src_p, dst_p)
    h1, g1 = _tc_combine(p0, x, Wr0, bl0r, Wl1, relu=True, project_next=True)
    p1 = _sc_aggregate(g1, src_p, dst_p)
    (out,) = _tc_combine(p1, h1, Wr1, bl1r, Wl1, relu=False,
                         project_next=False)
    return out
